# baseline (speedup 1.0000x reference)
import functools

import jax
import jax.numpy as jnp
from jax import lax
from jax.experimental import pallas as pl
from jax.experimental.pallas import tpu as pltpu
from jax.experimental.pallas import tpu_sc as plsc

_N = 10000
_E = 320000
_D = 128
_C = 64
_NC = 2
_NS = 16
_NW = _NC * _NS
_CH = 128
_NCHUNK = 80
_EPW = _CH * _NCHUNK
_EPAD = _EPW * _NW
_ROWS2D = _EPAD // _CH
_NP = 10240
_RPT = _NP // _NS


def _deg_body(dst2, out, dst_v, dcur_v, ones_v, zer_v, acc_sh):
    c = lax.axis_index("c")
    s = lax.axis_index("s")
    w = c * _NS + s
    one = jnp.ones((16,), jnp.float32)
    zero = jnp.zeros((16,), jnp.float32)

    def fill_ones(i, _):
        ones_v[i, :] = one
        return 0

    lax.fori_loop(0, _CH, fill_ones, 0)

    def fill_zeros(i, _):
        zer_v[i, :] = zero
        return 0

    lax.fori_loop(0, _RPT, fill_zeros, 0)
    pltpu.sync_copy(zer_v, acc_sh.at[pl.ds(s * _RPT, _RPT)])
    pltpu.sync_copy(dst2.at[pl.ds(w * _NCHUNK, _NCHUNK)], dst_v)
    plsc.subcore_barrier()

    def body(j, _):
        def cpidx(k, _2):
            dcur_v[pl.ds(k * 16, 16)] = dst_v[j, pl.ds(k * 16, 16)]
            return 0

        lax.fori_loop(0, _CH // 16, cpidx, 0)
        pltpu.sync_copy(ones_v, acc_sh.at[dcur_v], add=True)
        return 0

    lax.fori_loop(0, _NCHUNK, body, 0)
    plsc.subcore_barrier()
    ob = s * _RPT
    pltpu.sync_copy(acc_sh.at[pl.ds(ob, _RPT)], zer_v)
    pltpu.sync_copy(zer_v, out.at[c, pl.ds(ob, _RPT)])


_NOUT = _NCHUNK // 2


def _agg_body(hp, src2, dst2, out, sa0, da0, sa1, da1, rb0, rb1,
              acc_sh, gsem0, gsem1, isem0, isem1):
    c = lax.axis_index("c")
    s = lax.axis_index("s")
    w = c * _NS + s
    base_row = w * _NCHUNK
    zero = jnp.zeros((16,), jnp.float32)

    def zrow(i, _):
        def zcol(k, _2):
            rb0[i, pl.ds(k * 16, 16)] = zero
            return 0

        lax.fori_loop(0, _D // 16, zcol, 0)
        return 0

    lax.fori_loop(0, _CH, zrow, 0)
    base = s * _RPT
    for k in range(0, _RPT, _CH):
        pltpu.sync_copy(rb0, acc_sh.at[pl.ds(base + k, _CH)])
    sas = (sa0, sa1)
    das = (da0, da1)
    rbs = (rb0, rb1)
    gsems = (gsem0, gsem1)
    isems = (isem0, isem1)
    kb = len(rbs)
    for b in range(kb):
        pltpu.async_copy(src2.at[base_row + b], sas[b], isems[b])
        pltpu.async_copy(dst2.at[base_row + b], das[b], isems[b])
    plsc.subcore_barrier()
    ngrp = _NCHUNK // kb

    def outer(g, _):
        for b in range(kb):
            pltpu.make_async_copy(src2.at[0], sas[b], isems[b]).wait()
            pltpu.make_async_copy(dst2.at[0], das[b], isems[b]).wait()
            pltpu.async_copy(hp.at[sas[b]], rbs[b], gsems[b])
        for b in range(kb):
            pltpu.make_async_copy(hp.at[pl.ds(0, _CH)], rbs[b],
                                  gsems[b]).wait()
            pltpu.sync_copy(rbs[b], acc_sh.at[das[b]], add=True)

            @pl.when(g < ngrp - 1)
            def _(b=b):
                row = base_row + (g + 1) * kb + b
                pltpu.async_copy(src2.at[row], sas[b], isems[b])
                pltpu.async_copy(dst2.at[row], das[b], isems[b])

        return 0

    lax.fori_loop(0, ngrp, outer, 0)
    plsc.subcore_barrier()
    ob = s * _RPT
    for k in range(0, _RPT, _CH):
        pltpu.sync_copy(acc_sh.at[pl.ds(ob + k, _CH)], rb0)
        pltpu.sync_copy(rb0, out.at[c, pl.ds(ob + k, _CH)])


@functools.cache
def _get_deg_kernel():
    mesh = plsc.VectorSubcoreMesh(
        core_axis_name="c", subcore_axis_name="s",
        num_cores=_NC, num_subcores=_NS)
    return pl.kernel(
        _deg_body,
        out_type=jax.ShapeDtypeStruct((_NC, _NP, 16), jnp.float32),
        mesh=mesh,
        compiler_params=pltpu.CompilerParams(use_tc_tiling_on_sc=False),
        scratch_types=[
            pltpu.VMEM((_NCHUNK, _CH), jnp.int32),
            pltpu.VMEM((_CH,), jnp.int32),
            pltpu.VMEM((_CH, 16), jnp.float32),
            pltpu.VMEM((_RPT, 16), jnp.float32),
            pltpu.VMEM_SHARED((_NP, 16), jnp.float32),
        ],
    )


@functools.cache
def _get_agg_kernel():
    mesh = plsc.VectorSubcoreMesh(
        core_axis_name="c", subcore_axis_name="s",
        num_cores=_NC, num_subcores=_NS)
    return pl.kernel(
        _agg_body,
        out_type=jax.ShapeDtypeStruct((_NC, _NP, _D), jnp.float32),
        mesh=mesh,
        compiler_params=pltpu.CompilerParams(use_tc_tiling_on_sc=False),
        scratch_types=(
            [pltpu.VMEM((_CH,), jnp.int32)] * 4
            + [pltpu.VMEM((_CH, _D), jnp.float32)] * 2
            + [pltpu.VMEM_SHARED((_NP, _D), jnp.float32)]
            + [pltpu.SemaphoreType.DMA] * 4
        ),
    )


def _dinv_from(degp_ref):
    deg = degp_ref[0, 0:_N, 0:1] + degp_ref[1, 0:_N, 0:1] + 1.0
    return lax.rsqrt(jnp.maximum(deg, 1.0))


def _dot(a, b):
    return jnp.dot(a, b, preferred_element_type=jnp.float32,
                   precision=lax.Precision.HIGHEST)


def _tc1_body(x_ref, w1_ref, degp_ref, out_ref):
    dinv = _dinv_from(degp_ref)
    out_ref[...] = dinv * _dot(x_ref[...], w1_ref[...])


def _bn_relu(h, g_ref, be_ref):
    mu = jnp.mean(h, axis=0, keepdims=True)
    var = jnp.mean((h - mu) ** 2, axis=0, keepdims=True)
    return jnp.maximum((h - mu) * lax.rsqrt(var + 1e-5) * g_ref[...]
                       + be_ref[...], 0.0)


def _tc2_body(acc_ref, hp_ref, degp_ref, b_ref, g_ref, be_ref, w2_ref,
              out_ref):
    dinv = _dinv_from(degp_ref)
    h = dinv * (acc_ref[0, 0:_N] + acc_ref[1, 0:_N] + hp_ref[...]) + b_ref[...]
    r = _bn_relu(h, g_ref, be_ref)
    out_ref[...] = dinv * _dot(r, w2_ref[...])


def _tc3_body(acc_ref, hp_ref, degp_ref, b_ref, g_ref, be_ref,
              wf1_ref, bf1_ref, wf2_ref, bf2_ref, out_ref):
    dinv = _dinv_from(degp_ref)
    h = dinv * (acc_ref[0, 0:_N] + acc_ref[1, 0:_N] + hp_ref[...]) + b_ref[...]
    r = _bn_relu(h, g_ref, be_ref)
    m = jnp.maximum(_dot(r, wf1_ref[...]) + bf1_ref[...], 0.0)
    o = _dot(m, wf2_ref[...]) + bf2_ref[...]
    sh = o - jnp.max(o, axis=1, keepdims=True)
    out_ref[...] = sh - jnp.log(jnp.sum(jnp.exp(sh), axis=1, keepdims=True))


_tc1_call = pl.pallas_call(
    _tc1_body, out_shape=jax.ShapeDtypeStruct((_N, _D), jnp.float32))
_tc2_call = pl.pallas_call(
    _tc2_body, out_shape=jax.ShapeDtypeStruct((_N, _D), jnp.float32))
_tc3_call = pl.pallas_call(
    _tc3_body, out_shape=jax.ShapeDtypeStruct((_N, _C), jnp.float32))


def kernel(x, edge_index, W1, b1, g1, be1, W2, b2, g2, be2, Wf1, bf1,
           Wf2, bf2):
    src = edge_index[0].astype(jnp.int32)
    dst = edge_index[1].astype(jnp.int32)
    npad = _EPAD - _E
    src2 = jnp.concatenate(
        [src, jnp.zeros((npad,), jnp.int32)]).reshape(_ROWS2D, _CH)
    pad_dst = _N + jnp.arange(npad, dtype=jnp.int32) % (_NP - _N)
    dst2 = jnp.concatenate([dst, pad_dst]).reshape(_ROWS2D, _CH)
    degp = _get_deg_kernel()(dst2)
    hp1 = _tc1_call(x, W1, degp)
    acc1 = _get_agg_kernel()(hp1, src2, dst2)
    hp2 = _tc2_call(acc1, hp1, degp, b1.reshape(1, -1), g1.reshape(1, -1),
                    be1.reshape(1, -1), W2)
    acc2 = _get_agg_kernel()(hp2, src2, dst2)
    return _tc3_call(acc2, hp2, degp, b2.reshape(1, -1), g2.reshape(1, -1),
                     be2.reshape(1, -1), Wf1, bf1.reshape(1, -1),
                     Wf2, bf2.reshape(1, -1))


# speedup vs baseline: 2.8075x; 2.8075x over previous
import functools

import jax
import jax.numpy as jnp
from jax import lax
from jax.experimental import pallas as pl
from jax.experimental.pallas import tpu as pltpu
from jax.experimental.pallas import tpu_sc as plsc

_N = 10000
_E = 320000
_D = 128
_C = 64
_NC = 2
_NS = 16
_NW = _NC * _NS
_CH = 128
_NCHUNK = 80
_EPW = _CH * _NCHUNK
_EPAD = _EPW * _NW
_ROWS2D = _EPAD // _CH
_NP = 10240
_RPT = _NP // _NS


def _deg_body(dst2, out, dst_v, dcur_v, ones_v, zer_v, acc_sh):
    c = lax.axis_index("c")
    s = lax.axis_index("s")
    w = c * _NS + s
    one = jnp.ones((16,), jnp.float32)
    zero = jnp.zeros((16,), jnp.float32)

    def fill_ones(i, _):
        ones_v[i, :] = one
        return 0

    lax.fori_loop(0, _CH, fill_ones, 0)

    def fill_zeros(i, _):
        zer_v[i, :] = zero
        return 0

    lax.fori_loop(0, _RPT, fill_zeros, 0)
    pltpu.sync_copy(zer_v, acc_sh.at[pl.ds(s * _RPT, _RPT)])
    pltpu.sync_copy(dst2.at[pl.ds(w * _NCHUNK, _NCHUNK)], dst_v)
    plsc.subcore_barrier()

    def body(j, _):
        def cpidx(k, _2):
            dcur_v[pl.ds(k * 16, 16)] = dst_v[j, pl.ds(k * 16, 16)]
            return 0

        lax.fori_loop(0, _CH // 16, cpidx, 0)
        pltpu.sync_copy(ones_v, acc_sh.at[dcur_v], add=True)
        return 0

    lax.fori_loop(0, _NCHUNK, body, 0)
    plsc.subcore_barrier()
    ob = s * _RPT
    pltpu.sync_copy(acc_sh.at[pl.ds(ob, _RPT)], zer_v)
    pltpu.sync_copy(zer_v, out.at[c, pl.ds(ob, _RPT)])


_NOUT = _NCHUNK // 2


def _agg_body(hp, src2, dst2, out, sa0, da0, sa1, da1, rb0, rb1,
              acc_sh, gsem0, gsem1, isem0, isem1):
    c = lax.axis_index("c")
    s = lax.axis_index("s")
    w = c * _NS + s
    base_row = w * _NCHUNK
    zero = jnp.zeros((16,), jnp.float32)

    def zrow(i, _):
        def zcol(k, _2):
            rb0[i, pl.ds(k * 16, 16)] = zero
            return 0

        lax.fori_loop(0, _D // 16, zcol, 0)
        return 0

    lax.fori_loop(0, _CH, zrow, 0)
    base = s * _RPT
    for k in range(0, _RPT, _CH):
        pltpu.sync_copy(rb0, acc_sh.at[pl.ds(base + k, _CH)])
    sas = (sa0, sa1)
    das = (da0, da1)
    rbs = (rb0, rb1)
    gsems = (gsem0, gsem1)
    isems = (isem0, isem1)
    kb = len(rbs)
    for b in range(kb):
        pltpu.async_copy(src2.at[base_row + b], sas[b], isems[b])
        pltpu.async_copy(dst2.at[base_row + b], das[b], isems[b])
    plsc.subcore_barrier()
    ngrp = _NCHUNK // kb

    def outer(g, _):
        for b in range(kb):
            pltpu.make_async_copy(src2.at[0], sas[b], isems[b]).wait()
            pltpu.make_async_copy(dst2.at[0], das[b], isems[b]).wait()
            pltpu.async_copy(hp.at[sas[b]], rbs[b], gsems[b])
        for b in range(kb):
            pltpu.make_async_copy(hp.at[pl.ds(0, _CH)], rbs[b],
                                  gsems[b]).wait()
            pltpu.sync_copy(rbs[b], acc_sh.at[das[b]], add=True)

            @pl.when(g < ngrp - 1)
            def _(b=b):
                row = base_row + (g + 1) * kb + b
                pltpu.async_copy(src2.at[row], sas[b], isems[b])
                pltpu.async_copy(dst2.at[row], das[b], isems[b])

        return 0

    lax.fori_loop(0, ngrp, outer, 0)
    plsc.subcore_barrier()
    ob = s * _RPT
    for k in range(0, _RPT, _CH):
        pltpu.sync_copy(acc_sh.at[pl.ds(ob + k, _CH)], rb0)
        pltpu.sync_copy(rb0, out.at[c, pl.ds(ob + k, _CH)])


@functools.cache
def _get_deg_kernel():
    mesh = plsc.VectorSubcoreMesh(
        core_axis_name="c", subcore_axis_name="s",
        num_cores=_NC, num_subcores=_NS)
    return pl.kernel(
        _deg_body,
        out_type=jax.ShapeDtypeStruct((_NC, _NP, 16), jnp.float32),
        mesh=mesh,
        compiler_params=pltpu.CompilerParams(use_tc_tiling_on_sc=False),
        scratch_types=[
            pltpu.VMEM((_NCHUNK, _CH), jnp.int32),
            pltpu.VMEM((_CH,), jnp.int32),
            pltpu.VMEM((_CH, 16), jnp.float32),
            pltpu.VMEM((_RPT, 16), jnp.float32),
            pltpu.VMEM_SHARED((_NP, 16), jnp.float32),
        ],
    )


@functools.cache
def _get_agg_kernel():
    mesh = plsc.VectorSubcoreMesh(
        core_axis_name="c", subcore_axis_name="s",
        num_cores=_NC, num_subcores=_NS)
    return pl.kernel(
        _agg_body,
        out_type=jax.ShapeDtypeStruct((_NC, _NP, _D), jnp.float32),
        mesh=mesh,
        compiler_params=pltpu.CompilerParams(use_tc_tiling_on_sc=False),
        scratch_types=(
            [pltpu.VMEM((_CH,), jnp.int32)] * 4
            + [pltpu.VMEM((_CH, _D), jnp.float32)] * 2
            + [pltpu.VMEM_SHARED((_NP, _D), jnp.float32)]
            + [pltpu.SemaphoreType.DMA] * 4
        ),
    )


def _dinv_from(degp_ref):
    deg = degp_ref[0, 0:_N, 0:1] + degp_ref[1, 0:_N, 0:1] + 1.0
    return lax.rsqrt(jnp.maximum(deg, 1.0))


def _dot(a, b):
    return jnp.dot(a, b, preferred_element_type=jnp.float32,
                   precision=lax.Precision.HIGHEST)


def _tc1_body(x_ref, w1_ref, degp_ref, out_ref):
    dinv = _dinv_from(degp_ref)
    out_ref[...] = dinv * _dot(x_ref[...], w1_ref[...])


def _bn_relu(h, g_ref, be_ref):
    mu = jnp.mean(h, axis=0, keepdims=True)
    var = jnp.mean((h - mu) ** 2, axis=0, keepdims=True)
    return jnp.maximum((h - mu) * lax.rsqrt(var + 1e-5) * g_ref[...]
                       + be_ref[...], 0.0)


def _tc2_body(acc_ref, hp_ref, degp_ref, b_ref, g_ref, be_ref, w2_ref,
              out_ref):
    dinv = _dinv_from(degp_ref)
    h = dinv * (acc_ref[0, 0:_N] + acc_ref[1, 0:_N] + hp_ref[...]) + b_ref[...]
    r = _bn_relu(h, g_ref, be_ref)
    out_ref[...] = dinv * _dot(r, w2_ref[...])


def _tc3_body(acc_ref, hp_ref, degp_ref, b_ref, g_ref, be_ref,
              wf1_ref, bf1_ref, wf2_ref, bf2_ref, out_ref):
    dinv = _dinv_from(degp_ref)
    h = dinv * (acc_ref[0, 0:_N] + acc_ref[1, 0:_N] + hp_ref[...]) + b_ref[...]
    r = _bn_relu(h, g_ref, be_ref)
    m = jnp.maximum(_dot(r, wf1_ref[...]) + bf1_ref[...], 0.0)
    o = _dot(m, wf2_ref[...]) + bf2_ref[...]
    sh = o - jnp.max(o, axis=1, keepdims=True)
    out_ref[...] = sh - jnp.log(jnp.sum(jnp.exp(sh), axis=1, keepdims=True))


_tc1_call = pl.pallas_call(
    _tc1_body, out_shape=jax.ShapeDtypeStruct((_N, _D), jnp.float32))
_tc2_call = pl.pallas_call(
    _tc2_body, out_shape=jax.ShapeDtypeStruct((_N, _D), jnp.float32))
_tc3_call = pl.pallas_call(
    _tc3_body, out_shape=jax.ShapeDtypeStruct((_N, _C), jnp.float32))


def kernel(x, edge_index, W1, b1, g1, be1, W2, b2, g2, be2, Wf1, bf1,
           Wf2, bf2):
    src = edge_index[0].astype(jnp.int32)
    dst = edge_index[1].astype(jnp.int32)
    npad = _EPAD - _E
    pad_src = jnp.arange(npad, dtype=jnp.int32) % _N
    src2 = jnp.concatenate([src, pad_src]).reshape(_ROWS2D, _CH)
    pad_dst = _N + jnp.arange(npad, dtype=jnp.int32) % (_NP - _N)
    dst2 = jnp.concatenate([dst, pad_dst]).reshape(_ROWS2D, _CH)
    degp = _get_deg_kernel()(dst2)
    hp1 = _tc1_call(x, W1, degp)
    acc1 = _get_agg_kernel()(hp1, src2, dst2)
    hp2 = _tc2_call(acc1, hp1, degp, b1.reshape(1, -1), g1.reshape(1, -1),
                    be1.reshape(1, -1), W2)
    acc2 = _get_agg_kernel()(hp2, src2, dst2)
    return _tc3_call(acc2, hp2, degp, b2.reshape(1, -1), g2.reshape(1, -1),
                     be2.reshape(1, -1), Wf1, bf1.reshape(1, -1),
                     Wf2, bf2.reshape(1, -1))
